# baseline (speedup 1.0000x reference)
import jax
import jax.numpy as jnp
from jax import lax
from jax.experimental import pallas as pl
from jax.experimental.pallas import tpu as pltpu
from jax.experimental.pallas import tpu_sc as plsc

N = 10000
D = 128
DH = D // 2
E = 320000
CHUNK = 125
NC = 2
NS = 16
NW = NC * NS
EPW = E // NW
CPW = EPW // CHUNK
NROWS = E // CHUNK
SLAB = 624
TAIL = N - NS * SLAB
DEG_W = 16
ZC = 104

_f32 = jnp.float32


def _sc_mesh():
    return plsc.VectorSubcoreMesh(core_axis_name="c", subcore_axis_name="s")


def _fill(buf, rows, val):
    groups = buf.shape[1] // 16

    @pl.loop(0, rows)
    def _(r):
        @pl.loop(0, groups)
        def _(g):
            buf[r, pl.ds(g * 16, 16)] = jnp.full((16,), val, dtype=_f32)


def _zero_slab(zbuf, acc, s):
    @pl.loop(0, SLAB // ZC)
    def _(i):
        pltpu.sync_copy(zbuf.at[pl.ds(0, ZC)],
                        acc.at[pl.ds(s * SLAB + i * ZC, ZC)])

    @pl.when(s == NS - 1)
    def _():
        pltpu.sync_copy(zbuf.at[pl.ds(0, TAIL)],
                        acc.at[pl.ds(NS * SLAB, TAIL)])


def _drain_slab(acc, out_slab, s):
    pltpu.sync_copy(acc.at[pl.ds(s * SLAB, SLAB)],
                    out_slab.at[pl.ds(s * SLAB, SLAB)])

    @pl.when(s == NS - 1)
    def _():
        pltpu.sync_copy(acc.at[pl.ds(NS * SLAB, TAIL)],
                        out_slab.at[pl.ds(NS * SLAB, TAIL)])


def _sc_degree_body(dst_hbm, out_hbm, idx_v, ones_v, sem, acc):
    c = lax.axis_index("c")
    s = lax.axis_index("s")
    wid = c * NS + s
    pltpu.async_copy(dst_hbm.at[pl.ds(wid * CPW, CPW)], idx_v, sem).wait()
    _fill(ones_v, CHUNK, 0.0)
    _zero_slab(ones_v, acc, s)
    _fill(ones_v, CHUNK, 1.0)
    plsc.subcore_barrier()

    @pl.loop(0, CPW)
    def _(j):
        pltpu.sync_copy(ones_v, acc.at[idx_v.at[j]], add=True)

    plsc.subcore_barrier()
    _drain_slab(acc, out_hbm.at[c], s)


def _sc_degree(dst2d):
    kern = pl.kernel(
        _sc_degree_body,
        out_type=jax.ShapeDtypeStruct((NC, N, DEG_W), _f32),
        mesh=_sc_mesh(),
        scratch_types=[
            pltpu.VMEM((CPW, CHUNK), jnp.int32),
            pltpu.VMEM((CHUNK, DEG_W), _f32),
            pltpu.SemaphoreType.DMA,
            pltpu.VMEM_SHARED((N, DEG_W), _f32),
        ],
        compiler_params=pltpu.CompilerParams(use_tc_tiling_on_sc=False),
    )
    return kern(dst2d)


NBUF = 5
CPT = 2 * CPW


KCH = 128
KCPT = 157
KEPT = KCPT * KCH


def _sc_scatter_body(src_hbm, dst_hbm, xs_hbm, out_hbm,
                     srcv, dstv, bufs, gsems, ssems, acc):
    c = lax.axis_index("c")
    s = lax.axis_index("s")
    pltpu.async_copy(src_hbm.at[pl.ds(s * KCPT, KCPT)], srcv, gsems[0]).wait()
    pltpu.async_copy(dst_hbm.at[pl.ds(s * KCPT, KCPT)], dstv, gsems[0]).wait()
    _fill(bufs[0], ZC, 0.0)
    _zero_slab(bufs[0], acc, s)
    plsc.subcore_barrier()

    xs_half = xs_hbm.at[c]

    def batch(j0, cnt):
        gs = [pltpu.async_copy(xs_half.at[srcv.at[j0 + i]], bufs[i], gsems[i])
              for i in range(cnt)]
        ss = []
        for i in range(cnt):
            gs[i].wait()
            ss.append(pltpu.async_copy(bufs[i], acc.at[dstv.at[j0 + i]],
                                       ssems[i], add=True))
        for cp in ss:
            cp.wait()

    @pl.loop(0, KCPT // NBUF)
    def _(t):
        batch(NBUF * t, NBUF)

    if KCPT % NBUF:
        batch((KCPT // NBUF) * NBUF, KCPT % NBUF)

    plsc.subcore_barrier()
    _drain_slab(acc, out_hbm.at[c], s)


def _sc_scatter(src_pad, dst_pad, xs2n):
    kern = pl.kernel(
        _sc_scatter_body,
        out_type=jax.ShapeDtypeStruct((NC, N, DH), _f32),
        mesh=_sc_mesh(),
        scratch_types=[
            pltpu.VMEM((KCPT, KCH), jnp.int32),
            pltpu.VMEM((KCPT, KCH), jnp.int32),
            [pltpu.VMEM((KCH, DH), _f32) for _ in range(NBUF)],
            [pltpu.SemaphoreType.DMA for _ in range(NBUF)],
            [pltpu.SemaphoreType.DMA for _ in range(NBUF)],
            pltpu.VMEM_SHARED((N + NS, DH), _f32),
        ],
        compiler_params=pltpu.CompilerParams(use_tc_tiling_on_sc=False),
    )
    return kern(src_pad, dst_pad, xs2n)


ROWB = 2000


def _prep_body(deg_ref, x_ref, w_ref, xs2_ref, dinv_ref):
    deg = deg_ref[0, :, :1] + deg_ref[1, :, :1] + 1.0
    dinv = lax.rsqrt(deg)
    xw = jnp.dot(x_ref[...], w_ref[...], preferred_element_type=_f32)
    xs = xw * dinv
    xs2_ref[0] = xs[:, :DH]
    xs2_ref[1] = xs[:, DH:]
    dinv_ref[...] = dinv


def _prep(deg_parts, x, W0):
    return pl.pallas_call(
        _prep_body,
        grid=(N // ROWB,),
        in_specs=[
            pl.BlockSpec((2, ROWB, DEG_W), lambda i: (0, i, 0)),
            pl.BlockSpec((ROWB, D), lambda i: (i, 0)),
            pl.BlockSpec((D, D), lambda i: (0, 0)),
        ],
        out_specs=[
            pl.BlockSpec((2, ROWB, DH), lambda i: (0, i, 0)),
            pl.BlockSpec((ROWB, 1), lambda i: (i, 0)),
        ],
        out_shape=[
            jax.ShapeDtypeStruct((2, N, DH), _f32),
            jax.ShapeDtypeStruct((N, 1), _f32),
        ],
    )(deg_parts, x, W0)


def _conv_out(p_ref, xs2_ref, dinv, b_ref):
    ylo = (p_ref[0] + xs2_ref[0]) * dinv + b_ref[...][:, :DH]
    yhi = (p_ref[1] + xs2_ref[1]) * dinv + b_ref[...][:, DH:]
    return jnp.concatenate([ylo, yhi], axis=1)


def _mid_body(p_ref, xs2_ref, dinv_ref, res_ref, b_ref, w_ref,
              h_ref, xsn2_ref):
    dinv = dinv_ref[...]
    y = _conv_out(p_ref, xs2_ref, dinv, b_ref)
    h = jnp.maximum(y, 0.0) + res_ref[...]
    h_ref[...] = h
    xsn = jnp.dot(h, w_ref[...], preferred_element_type=_f32) * dinv
    xsn2_ref[0] = xsn[:, :DH]
    xsn2_ref[1] = xsn[:, DH:]


def _mid(parts, xs2, dinv, res, b, Wn):
    return pl.pallas_call(
        _mid_body,
        grid=(N // ROWB,),
        in_specs=[
            pl.BlockSpec((2, ROWB, DH), lambda i: (0, i, 0)),
            pl.BlockSpec((2, ROWB, DH), lambda i: (0, i, 0)),
            pl.BlockSpec((ROWB, 1), lambda i: (i, 0)),
            pl.BlockSpec((ROWB, D), lambda i: (i, 0)),
            pl.BlockSpec((1, D), lambda i: (0, 0)),
            pl.BlockSpec((D, D), lambda i: (0, 0)),
        ],
        out_specs=[
            pl.BlockSpec((ROWB, D), lambda i: (i, 0)),
            pl.BlockSpec((2, ROWB, DH), lambda i: (0, i, 0)),
        ],
        out_shape=[
            jax.ShapeDtypeStruct((N, D), _f32),
            jax.ShapeDtypeStruct((2, N, DH), _f32),
        ],
    )(parts, xs2, dinv, res, b.reshape(1, D), Wn)


def _final_body(p_ref, xs2_ref, dinv_ref, b_ref, out_ref):
    y = _conv_out(p_ref, xs2_ref, dinv_ref[...], b_ref)
    m = jnp.max(y, axis=1, keepdims=True)
    lse = jnp.log(jnp.sum(jnp.exp(y - m), axis=1, keepdims=True)) + m
    out_ref[...] = y - lse


def _final(parts, xs2, dinv, b):
    return pl.pallas_call(
        _final_body,
        grid=(N // ROWB,),
        in_specs=[
            pl.BlockSpec((2, ROWB, DH), lambda i: (0, i, 0)),
            pl.BlockSpec((2, ROWB, DH), lambda i: (0, i, 0)),
            pl.BlockSpec((ROWB, 1), lambda i: (i, 0)),
            pl.BlockSpec((1, D), lambda i: (0, 0)),
        ],
        out_specs=pl.BlockSpec((ROWB, D), lambda i: (i, 0)),
        out_shape=jax.ShapeDtypeStruct((N, D), _f32),
    )(parts, xs2, dinv, b.reshape(1, D))


def _pad_edges(vals, fill):
    v = vals.reshape(NS, E // NS)
    pad = jnp.broadcast_to(fill, (NS, KEPT - E // NS)).astype(v.dtype)
    return jnp.concatenate([v, pad], axis=1).reshape(NS * KCPT, KCH)


def kernel(x, edge_index, W0, b0, W1, b1, W2, b2):
    src_pad = _pad_edges(edge_index[0], jnp.zeros((NS, 1), jnp.int32))
    dst_pad = _pad_edges(edge_index[1],
                         N + jnp.arange(NS, dtype=jnp.int32)[:, None])
    dst2d = edge_index[1].reshape(NROWS, CHUNK)
    deg_parts = _sc_degree(dst2d)
    xs0, dinv = _prep(deg_parts, x, W0)
    p0 = _sc_scatter(src_pad, dst_pad, xs0)
    h1, xs1 = _mid(p0, xs0, dinv, x, b0, W1)
    p1 = _sc_scatter(src_pad, dst_pad, xs1)
    h2, xs2 = _mid(p1, xs1, dinv, h1, b1, W2)
    p2 = _sc_scatter(src_pad, dst_pad, xs2)
    return _final(p2, xs2, dinv, b2)


# speedup vs baseline: 1.1310x; 1.1310x over previous
import jax
import jax.numpy as jnp
from jax import lax
from jax.experimental import pallas as pl
from jax.experimental.pallas import tpu as pltpu
from jax.experimental.pallas import tpu_sc as plsc

N = 10000
D = 128
DH = D // 2
E = 320000
CHUNK = 125
NC = 2
NS = 16
NW = NC * NS
EPW = E // NW
CPW = EPW // CHUNK
NROWS = E // CHUNK
SLAB = 624
TAIL = N - NS * SLAB
DEG_W = 16
ZC = 104

_f32 = jnp.float32


def _sc_mesh():
    return plsc.VectorSubcoreMesh(core_axis_name="c", subcore_axis_name="s")


def _fill(buf, rows, val):
    groups = buf.shape[1] // 16

    @pl.loop(0, rows)
    def _(r):
        @pl.loop(0, groups)
        def _(g):
            buf[r, pl.ds(g * 16, 16)] = jnp.full((16,), val, dtype=_f32)


def _zero_slab(zbuf, acc, s):
    @pl.loop(0, SLAB // ZC)
    def _(i):
        pltpu.sync_copy(zbuf.at[pl.ds(0, ZC)],
                        acc.at[pl.ds(s * SLAB + i * ZC, ZC)])

    @pl.when(s == NS - 1)
    def _():
        pltpu.sync_copy(zbuf.at[pl.ds(0, TAIL)],
                        acc.at[pl.ds(NS * SLAB, TAIL)])


def _drain_slab(acc, out_slab, s):
    pltpu.sync_copy(acc.at[pl.ds(s * SLAB, SLAB)],
                    out_slab.at[pl.ds(s * SLAB, SLAB)])

    @pl.when(s == NS - 1)
    def _():
        pltpu.sync_copy(acc.at[pl.ds(NS * SLAB, TAIL)],
                        out_slab.at[pl.ds(NS * SLAB, TAIL)])


def _sc_degree_body(dst_hbm, out_hbm, idx_v, ones_v, sem, acc):
    c = lax.axis_index("c")
    s = lax.axis_index("s")
    wid = c * NS + s
    pltpu.async_copy(dst_hbm.at[pl.ds(wid * CPW, CPW)], idx_v, sem).wait()
    _fill(ones_v, CHUNK, 0.0)
    _zero_slab(ones_v, acc, s)
    _fill(ones_v, CHUNK, 1.0)
    plsc.subcore_barrier()

    @pl.loop(0, CPW)
    def _(j):
        pltpu.sync_copy(ones_v, acc.at[idx_v.at[j]], add=True)

    plsc.subcore_barrier()
    _drain_slab(acc, out_hbm.at[c], s)


def _sc_degree(dst2d):
    kern = pl.kernel(
        _sc_degree_body,
        out_type=jax.ShapeDtypeStruct((NC, N, DEG_W), _f32),
        mesh=_sc_mesh(),
        scratch_types=[
            pltpu.VMEM((CPW, CHUNK), jnp.int32),
            pltpu.VMEM((CHUNK, DEG_W), _f32),
            pltpu.SemaphoreType.DMA,
            pltpu.VMEM_SHARED((N, DEG_W), _f32),
        ],
        compiler_params=pltpu.CompilerParams(use_tc_tiling_on_sc=False),
    )
    return kern(dst2d)


NBUF = 5
CPT = 2 * CPW


def _sc_scatter_body(src_hbm, dst_hbm, xs_hbm, out_hbm,
                     srcv, dstv, bufs, gsems, ssems, acc):
    c = lax.axis_index("c")
    s = lax.axis_index("s")
    pltpu.async_copy(src_hbm.at[pl.ds(s * CPT, CPT)], srcv, gsems[0]).wait()
    pltpu.async_copy(dst_hbm.at[pl.ds(s * CPT, CPT)], dstv, gsems[0]).wait()
    _fill(bufs[0], ZC, 0.0)
    _zero_slab(bufs[0], acc, s)
    plsc.subcore_barrier()

    xs_half = xs_hbm.at[c]

    @pl.loop(0, CPT // NBUF)
    def _(t):
        j0 = NBUF * t
        gs = [pltpu.async_copy(xs_half.at[srcv.at[j0 + i]], bufs[i],
                               gsems[i])
              for i in range(NBUF)]
        ss = []
        for i in range(NBUF):
            gs[i].wait()
            ss.append(pltpu.async_copy(bufs[i], acc.at[dstv.at[j0 + i]],
                                       ssems[i], add=True))
        for cp in ss:
            cp.wait()

    plsc.subcore_barrier()
    _drain_slab(acc, out_hbm.at[c], s)


def _sc_scatter(src2d, dst2d, xs2):
    kern = pl.kernel(
        _sc_scatter_body,
        out_type=jax.ShapeDtypeStruct((NC, N, DH), _f32),
        mesh=_sc_mesh(),
        scratch_types=[
            pltpu.VMEM((CPT, CHUNK), jnp.int32),
            pltpu.VMEM((CPT, CHUNK), jnp.int32),
            [pltpu.VMEM((CHUNK, DH), _f32) for _ in range(NBUF)],
            [pltpu.SemaphoreType.DMA for _ in range(NBUF)],
            [pltpu.SemaphoreType.DMA for _ in range(NBUF)],
            pltpu.VMEM_SHARED((N, DH), _f32),
        ],
        compiler_params=pltpu.CompilerParams(use_tc_tiling_on_sc=False),
    )
    return kern(src2d, dst2d, xs2)


ROWB = 2000


def _prep_body(deg_ref, x_ref, w_ref, xs2_ref, dinv_ref):
    deg = deg_ref[0, :, :1] + deg_ref[1, :, :1] + 1.0
    dinv = lax.rsqrt(deg)
    xw = jnp.dot(x_ref[...], w_ref[...], preferred_element_type=_f32)
    xs = xw * dinv
    xs2_ref[0] = xs[:, :DH]
    xs2_ref[1] = xs[:, DH:]
    dinv_ref[...] = dinv


def _prep(deg_parts, x, W0):
    return pl.pallas_call(
        _prep_body,
        grid=(N // ROWB,),
        in_specs=[
            pl.BlockSpec((2, ROWB, DEG_W), lambda i: (0, i, 0)),
            pl.BlockSpec((ROWB, D), lambda i: (i, 0)),
            pl.BlockSpec((D, D), lambda i: (0, 0)),
        ],
        out_specs=[
            pl.BlockSpec((2, ROWB, DH), lambda i: (0, i, 0)),
            pl.BlockSpec((ROWB, 1), lambda i: (i, 0)),
        ],
        out_shape=[
            jax.ShapeDtypeStruct((2, N, DH), _f32),
            jax.ShapeDtypeStruct((N, 1), _f32),
        ],
    )(deg_parts, x, W0)


def _conv_out(p_ref, xs2_ref, dinv, b_ref):
    ylo = (p_ref[0] + xs2_ref[0]) * dinv + b_ref[...][:, :DH]
    yhi = (p_ref[1] + xs2_ref[1]) * dinv + b_ref[...][:, DH:]
    return jnp.concatenate([ylo, yhi], axis=1)


def _mid_body(p_ref, xs2_ref, dinv_ref, res_ref, b_ref, w_ref,
              h_ref, xsn2_ref):
    dinv = dinv_ref[...]
    y = _conv_out(p_ref, xs2_ref, dinv, b_ref)
    h = jnp.maximum(y, 0.0) + res_ref[...]
    h_ref[...] = h
    xsn = jnp.dot(h, w_ref[...], preferred_element_type=_f32) * dinv
    xsn2_ref[0] = xsn[:, :DH]
    xsn2_ref[1] = xsn[:, DH:]


def _mid(parts, xs2, dinv, res, b, Wn):
    return pl.pallas_call(
        _mid_body,
        grid=(N // ROWB,),
        in_specs=[
            pl.BlockSpec((2, ROWB, DH), lambda i: (0, i, 0)),
            pl.BlockSpec((2, ROWB, DH), lambda i: (0, i, 0)),
            pl.BlockSpec((ROWB, 1), lambda i: (i, 0)),
            pl.BlockSpec((ROWB, D), lambda i: (i, 0)),
            pl.BlockSpec((1, D), lambda i: (0, 0)),
            pl.BlockSpec((D, D), lambda i: (0, 0)),
        ],
        out_specs=[
            pl.BlockSpec((ROWB, D), lambda i: (i, 0)),
            pl.BlockSpec((2, ROWB, DH), lambda i: (0, i, 0)),
        ],
        out_shape=[
            jax.ShapeDtypeStruct((N, D), _f32),
            jax.ShapeDtypeStruct((2, N, DH), _f32),
        ],
    )(parts, xs2, dinv, res, b.reshape(1, D), Wn)


def _final_body(p_ref, xs2_ref, dinv_ref, b_ref, out_ref):
    y = _conv_out(p_ref, xs2_ref, dinv_ref[...], b_ref)
    m = jnp.max(y, axis=1, keepdims=True)
    lse = jnp.log(jnp.sum(jnp.exp(y - m), axis=1, keepdims=True)) + m
    out_ref[...] = y - lse


def _final(parts, xs2, dinv, b):
    return pl.pallas_call(
        _final_body,
        grid=(N // ROWB,),
        in_specs=[
            pl.BlockSpec((2, ROWB, DH), lambda i: (0, i, 0)),
            pl.BlockSpec((2, ROWB, DH), lambda i: (0, i, 0)),
            pl.BlockSpec((ROWB, 1), lambda i: (i, 0)),
            pl.BlockSpec((1, D), lambda i: (0, 0)),
        ],
        out_specs=pl.BlockSpec((ROWB, D), lambda i: (i, 0)),
        out_shape=jax.ShapeDtypeStruct((N, D), _f32),
    )(parts, xs2, dinv, b.reshape(1, D))


def kernel(x, edge_index, W0, b0, W1, b1, W2, b2):
    src2d = edge_index[0].reshape(NROWS, CHUNK)
    dst2d = edge_index[1].reshape(NROWS, CHUNK)
    deg_parts = _sc_degree(dst2d)
    xs0, dinv = _prep(deg_parts, x, W0)
    p0 = _sc_scatter(src2d, dst2d, xs0)
    h1, xs1 = _mid(p0, xs0, dinv, x, b0, W1)
    p1 = _sc_scatter(src2d, dst2d, xs1)
    h2, xs2 = _mid(p1, xs1, dinv, h1, b1, W2)
    p2 = _sc_scatter(src2d, dst2d, xs2)
    return _final(p2, xs2, dinv, b2)


# speedup vs baseline: 1.1487x; 1.0156x over previous
import jax
import jax.numpy as jnp
from jax import lax
from jax.experimental import pallas as pl
from jax.experimental.pallas import tpu as pltpu
from jax.experimental.pallas import tpu_sc as plsc

N = 10000
D = 128
DH = D // 2
E = 320000
CHUNK = 125
NC = 2
NS = 16
NW = NC * NS
EPW = E // NW
CPW = EPW // CHUNK
NROWS = E // CHUNK
SLAB = 624
TAIL = N - NS * SLAB
DEG_W = 16
ZC = 104

_f32 = jnp.float32


def _sc_mesh():
    return plsc.VectorSubcoreMesh(core_axis_name="c", subcore_axis_name="s")


def _fill(buf, rows, val):
    groups = buf.shape[1] // 16

    @pl.loop(0, rows)
    def _(r):
        @pl.loop(0, groups)
        def _(g):
            buf[r, pl.ds(g * 16, 16)] = jnp.full((16,), val, dtype=_f32)


def _zero_slab(zbuf, acc, s):
    @pl.loop(0, SLAB // ZC)
    def _(i):
        pltpu.sync_copy(zbuf.at[pl.ds(0, ZC)],
                        acc.at[pl.ds(s * SLAB + i * ZC, ZC)])

    @pl.when(s == NS - 1)
    def _():
        pltpu.sync_copy(zbuf.at[pl.ds(0, TAIL)],
                        acc.at[pl.ds(NS * SLAB, TAIL)])


def _drain_slab(acc, out_slab, s):
    pltpu.sync_copy(acc.at[pl.ds(s * SLAB, SLAB)],
                    out_slab.at[pl.ds(s * SLAB, SLAB)])

    @pl.when(s == NS - 1)
    def _():
        pltpu.sync_copy(acc.at[pl.ds(NS * SLAB, TAIL)],
                        out_slab.at[pl.ds(NS * SLAB, TAIL)])


def _sc_degree_body(dst_hbm, out_hbm, idx_v, ones_v, sem, acc):
    c = lax.axis_index("c")
    s = lax.axis_index("s")
    wid = c * NS + s
    pltpu.async_copy(dst_hbm.at[pl.ds(wid * CPW, CPW)], idx_v, sem).wait()
    _fill(ones_v, CHUNK, 0.0)
    _zero_slab(ones_v, acc, s)
    _fill(ones_v, CHUNK, 1.0)
    plsc.subcore_barrier()

    @pl.loop(0, CPW)
    def _(j):
        pltpu.sync_copy(ones_v, acc.at[idx_v.at[j]], add=True)

    plsc.subcore_barrier()
    _drain_slab(acc, out_hbm.at[c], s)


def _sc_degree(dst2d):
    kern = pl.kernel(
        _sc_degree_body,
        out_type=jax.ShapeDtypeStruct((NC, N, DEG_W), _f32),
        mesh=_sc_mesh(),
        scratch_types=[
            pltpu.VMEM((CPW, CHUNK), jnp.int32),
            pltpu.VMEM((CHUNK, DEG_W), _f32),
            pltpu.SemaphoreType.DMA,
            pltpu.VMEM_SHARED((N, DEG_W), _f32),
        ],
        compiler_params=pltpu.CompilerParams(use_tc_tiling_on_sc=False),
    )
    return kern(dst2d)


NBUF = 5
CPT = 2 * CPW


def _sc_scatter_body(src_hbm, dst_hbm, xs_hbm, out_hbm,
                     srcv, dstv, bufs, gsems, ssems, acc):
    c = lax.axis_index("c")
    s = lax.axis_index("s")
    cp_s = pltpu.async_copy(src_hbm.at[pl.ds(s * CPT, CPT)], srcv, ssems[0])
    cp_d = pltpu.async_copy(dst_hbm.at[pl.ds(s * CPT, CPT)], dstv, ssems[1])
    _fill(bufs[0], ZC, 0.0)
    _zero_slab(bufs[0], acc, s)
    cp_s.wait()
    cp_d.wait()
    plsc.subcore_barrier()

    xs_half = xs_hbm.at[c]

    @pl.loop(0, CPT // NBUF)
    def _(t):
        j0 = NBUF * t
        gs = [pltpu.async_copy(xs_half.at[srcv.at[j0 + i]], bufs[i],
                               gsems[i])
              for i in range(NBUF)]
        ss = []
        for i in range(NBUF):
            gs[i].wait()
            ss.append(pltpu.async_copy(bufs[i], acc.at[dstv.at[j0 + i]],
                                       ssems[i], add=True))
        for cp in ss:
            cp.wait()

    plsc.subcore_barrier()
    _drain_slab(acc, out_hbm.at[c], s)


def _sc_scatter(src2d, dst2d, xs2):
    kern = pl.kernel(
        _sc_scatter_body,
        out_type=jax.ShapeDtypeStruct((NC, N, DH), _f32),
        mesh=_sc_mesh(),
        scratch_types=[
            pltpu.VMEM((CPT, CHUNK), jnp.int32),
            pltpu.VMEM((CPT, CHUNK), jnp.int32),
            [pltpu.VMEM((CHUNK, DH), _f32) for _ in range(NBUF)],
            [pltpu.SemaphoreType.DMA for _ in range(NBUF)],
            [pltpu.SemaphoreType.DMA for _ in range(NBUF)],
            pltpu.VMEM_SHARED((N, DH), _f32),
        ],
        compiler_params=pltpu.CompilerParams(use_tc_tiling_on_sc=False),
    )
    return kern(src2d, dst2d, xs2)


ROWB = 2000


def _prep_body(deg_ref, x_ref, w_ref, xs2_ref, dinv_ref):
    deg = deg_ref[0, :, :1] + deg_ref[1, :, :1] + 1.0
    dinv = lax.rsqrt(deg)
    xw = jnp.dot(x_ref[...], w_ref[...], preferred_element_type=_f32)
    xs = xw * dinv
    xs2_ref[0] = xs[:, :DH]
    xs2_ref[1] = xs[:, DH:]
    dinv_ref[...] = dinv


def _prep(deg_parts, x, W0):
    return pl.pallas_call(
        _prep_body,
        grid=(N // ROWB,),
        in_specs=[
            pl.BlockSpec((2, ROWB, DEG_W), lambda i: (0, i, 0)),
            pl.BlockSpec((ROWB, D), lambda i: (i, 0)),
            pl.BlockSpec((D, D), lambda i: (0, 0)),
        ],
        out_specs=[
            pl.BlockSpec((2, ROWB, DH), lambda i: (0, i, 0)),
            pl.BlockSpec((ROWB, 1), lambda i: (i, 0)),
        ],
        out_shape=[
            jax.ShapeDtypeStruct((2, N, DH), _f32),
            jax.ShapeDtypeStruct((N, 1), _f32),
        ],
    )(deg_parts, x, W0)


def _conv_out(p_ref, xs2_ref, dinv, b_ref):
    ylo = (p_ref[0] + xs2_ref[0]) * dinv + b_ref[...][:, :DH]
    yhi = (p_ref[1] + xs2_ref[1]) * dinv + b_ref[...][:, DH:]
    return jnp.concatenate([ylo, yhi], axis=1)


def _mid_body(p_ref, xs2_ref, dinv_ref, res_ref, b_ref, w_ref,
              h_ref, xsn2_ref):
    dinv = dinv_ref[...]
    y = _conv_out(p_ref, xs2_ref, dinv, b_ref)
    h = jnp.maximum(y, 0.0) + res_ref[...]
    h_ref[...] = h
    xsn = jnp.dot(h, w_ref[...], preferred_element_type=_f32) * dinv
    xsn2_ref[0] = xsn[:, :DH]
    xsn2_ref[1] = xsn[:, DH:]


def _mid(parts, xs2, dinv, res, b, Wn):
    return pl.pallas_call(
        _mid_body,
        grid=(N // ROWB,),
        in_specs=[
            pl.BlockSpec((2, ROWB, DH), lambda i: (0, i, 0)),
            pl.BlockSpec((2, ROWB, DH), lambda i: (0, i, 0)),
            pl.BlockSpec((ROWB, 1), lambda i: (i, 0)),
            pl.BlockSpec((ROWB, D), lambda i: (i, 0)),
            pl.BlockSpec((1, D), lambda i: (0, 0)),
            pl.BlockSpec((D, D), lambda i: (0, 0)),
        ],
        out_specs=[
            pl.BlockSpec((ROWB, D), lambda i: (i, 0)),
            pl.BlockSpec((2, ROWB, DH), lambda i: (0, i, 0)),
        ],
        out_shape=[
            jax.ShapeDtypeStruct((N, D), _f32),
            jax.ShapeDtypeStruct((2, N, DH), _f32),
        ],
    )(parts, xs2, dinv, res, b.reshape(1, D), Wn)


def _final_body(p_ref, xs2_ref, dinv_ref, b_ref, out_ref):
    y = _conv_out(p_ref, xs2_ref, dinv_ref[...], b_ref)
    m = jnp.max(y, axis=1, keepdims=True)
    lse = jnp.log(jnp.sum(jnp.exp(y - m), axis=1, keepdims=True)) + m
    out_ref[...] = y - lse


def _final(parts, xs2, dinv, b):
    return pl.pallas_call(
        _final_body,
        grid=(N // ROWB,),
        in_specs=[
            pl.BlockSpec((2, ROWB, DH), lambda i: (0, i, 0)),
            pl.BlockSpec((2, ROWB, DH), lambda i: (0, i, 0)),
            pl.BlockSpec((ROWB, 1), lambda i: (i, 0)),
            pl.BlockSpec((1, D), lambda i: (0, 0)),
        ],
        out_specs=pl.BlockSpec((ROWB, D), lambda i: (i, 0)),
        out_shape=jax.ShapeDtypeStruct((N, D), _f32),
    )(parts, xs2, dinv, b.reshape(1, D))


def kernel(x, edge_index, W0, b0, W1, b1, W2, b2):
    src2d = edge_index[0].reshape(NROWS, CHUNK)
    dst2d = edge_index[1].reshape(NROWS, CHUNK)
    deg_parts = _sc_degree(dst2d)
    xs0, dinv = _prep(deg_parts, x, W0)
    p0 = _sc_scatter(src2d, dst2d, xs0)
    h1, xs1 = _mid(p0, xs0, dinv, x, b0, W1)
    p1 = _sc_scatter(src2d, dst2d, xs1)
    h2, xs2 = _mid(p1, xs1, dinv, h1, b1, W2)
    p2 = _sc_scatter(src2d, dst2d, xs2)
    return _final(p2, xs2, dinv, b2)
